# baseline (speedup 1.0000x reference)
import jax
import jax.numpy as jnp
from jax import lax
from jax.experimental import pallas as pl
from jax.experimental.pallas import tpu as pltpu
from jax.experimental.pallas import tpu_sc as plsc

B, S, D = 4, 2048, 512
P = 16
MAXLEN = 127
WIN = 144
HALF = D // 2
LANES = 16
NHALF = HALF // LANES
SC_B = 3


def _sc_body(batch_hbm, len_hbm, out_hbm, len2d, len_v, buf, outbuf,
             out_sh, sem0, sem1):
  c_ax = lax.axis_index("c")
  s_ax = lax.axis_index("s")
  p_local = s_ax // 2
  p = c_ax * 8 + p_local
  half = s_ax % 2
  hoff = half * HALF

  pltpu.sync_copy(len_hbm, len2d)
  len_v[pl.ds(0, P)] = len2d[SC_B, pl.ds(0, P)]
  len_v[pl.ds(P, P)] = jnp.zeros((P,), jnp.int32)

  def lane0(j):
    return len_v[pl.ds(j, LANES)][0]

  begin = lax.fori_loop(0, p, lambda j, s: s + lane0(j), 0)
  length = lane0(p)

  mis = lax.rem(begin, 8)
  aligned = pl.multiple_of(begin - mis, 8)
  n = mis + length
  h = WIN // 2

  d0 = pltpu.make_async_copy(
      batch_hbm.at[SC_B, pl.ds(aligned, h), pl.ds(hoff, HALF)],
      buf.at[pl.ds(0, h)], sem0)
  d1 = pltpu.make_async_copy(
      batch_hbm.at[SC_B, pl.ds(aligned + h, h), pl.ds(hoff, HALF)],
      buf.at[pl.ds(h, h)], sem1)
  d0.start()

  @pl.when(n > h)
  def _():
    d1.start()

  zeros = tuple(jnp.zeros((LANES,), jnp.float32) for _ in range(NHALF))

  def body(r, accs):
    return tuple(accs[i] + buf[r, pl.ds(i * LANES, LANES)]
                 for i in range(NHALF))

  d0.wait()
  accs = lax.fori_loop(mis, jnp.minimum(n, h), body, zeros)

  @pl.when(n > h)
  def _():
    d1.wait()
  accs = lax.fori_loop(h, jnp.maximum(n, h), body, accs)
  denom = jnp.maximum(length, 1).astype(jnp.float32)
  for i in range(NHALF):
    outbuf[0, pl.ds(i * LANES, LANES)] = accs[i]

  def fix_chunk(i, _):
    v = outbuf[0, pl.ds(i * LANES, LANES)] / denom
    v = jnp.where(v == 0.0, jnp.full((LANES,), -1.0, jnp.float32), v)
    outbuf[0, pl.ds(i * LANES, LANES)] = v
    return 0
  lax.fori_loop(0, NHALF, fix_chunk, 0)
  pltpu.sync_copy(outbuf, out_sh.at[pl.ds(p_local, 1), pl.ds(hoff, HALF)])

  plsc.subcore_barrier()

  @pl.when(s_ax == 0)
  def _():
    pltpu.sync_copy(out_sh,
                    out_hbm.at[0, pl.ds(pl.multiple_of(c_ax * 8, 8), 8), :])


def _tc_body(len_ref, batch_ref, out_ref):
  pos = jax.lax.broadcasted_iota(jnp.int32, (1, S), 1)
  rows = []
  denoms = []
  cum = jnp.int32(0)
  one = jnp.ones((1, 1), jnp.float32)
  for p in range(P):
    ln = len_ref[0, 0, p]
    begin = cum
    cum = cum + ln
    rows.append(((pos >= begin) & (pos < cum)).astype(jnp.float32))
    denoms.append(one * jnp.maximum(ln, 1).astype(jnp.float32))
  mask = jnp.concatenate(rows, axis=0)
  denom = jnp.concatenate(denoms, axis=0)
  maskb = mask.astype(jnp.bfloat16)
  bf = batch_ref[...]
  bhi = bf.astype(jnp.bfloat16)
  blo = (bf - bhi.astype(jnp.float32)).astype(jnp.bfloat16)
  dn = (((1,), (0,)), ((), ()))
  acc = (jax.lax.dot_general(maskb, bhi, dimension_numbers=dn,
                             preferred_element_type=jnp.float32)
         + jax.lax.dot_general(maskb, blo, dimension_numbers=dn,
                               preferred_element_type=jnp.float32))
  res = acc / denom
  out_ref[...] = jnp.where(res == 0.0, -1.0, res)


@jax.jit
def kernel(batch, patch_lengths):
  lengths = patch_lengths
  if lengths.dtype != jnp.int32:
    lengths = lengths.astype(jnp.int32)

  mesh = plsc.VectorSubcoreMesh(core_axis_name="c", subcore_axis_name="s")
  sc_run = pl.kernel(
      _sc_body,
      out_type=jax.ShapeDtypeStruct((1, P, D), jnp.float32),
      mesh=mesh,
      scratch_types=[
          pltpu.VMEM((B, P), jnp.int32),
          pltpu.VMEM((2 * P,), jnp.int32),
          pltpu.VMEM((WIN, HALF), jnp.float32),
          pltpu.VMEM((1, HALF), jnp.float32),
          pltpu.VMEM_SHARED((8, D), jnp.float32),
          pltpu.SemaphoreType.DMA,
          pltpu.SemaphoreType.DMA,
      ],
  )
  sc_out = sc_run(batch, lengths)

  tc_fn = pl.pallas_call(
      lambda len_ref, batch_ref, out_ref: _tc_body(
          len_ref, batch_ref.at[0], out_ref.at[0]),
      grid=(B - 1,),
      in_specs=[
          pl.BlockSpec((1, 1, P), lambda b: (b, 0, 0),
                       memory_space=pltpu.SMEM),
          pl.BlockSpec((1, S, D), lambda b: (b, 0, 0)),
      ],
      out_specs=pl.BlockSpec((1, P, D), lambda b: (b, 0, 0)),
      out_shape=jax.ShapeDtypeStruct((B - 1, P, D), jnp.float32),
  )
  tc_out = tc_fn(lengths.reshape(B, 1, P), batch)

  return jnp.concatenate([tc_out, sc_out], axis=0)


# speedup vs baseline: 1.0093x; 1.0093x over previous
import jax
import jax.numpy as jnp
from jax import lax
from jax.experimental import pallas as pl
from jax.experimental.pallas import tpu as pltpu
from jax.experimental.pallas import tpu_sc as plsc

B, S, D = 4, 2048, 512
P = 16
MAXLEN = 127
WIN = 144
HALF = D // 2
LANES = 16
NHALF = HALF // LANES
SC_B = 3


def _sc_body(batch_hbm, len_hbm, out_hbm, len2d, len_v, buf, outbuf,
             out_sh, sem0, sem1):
  c_ax = lax.axis_index("c")
  s_ax = lax.axis_index("s")
  p_local = s_ax // 2
  p = c_ax * 8 + p_local
  half = s_ax % 2
  hoff = half * HALF

  pltpu.sync_copy(len_hbm, len2d)
  len_v[pl.ds(0, P)] = len2d[SC_B, pl.ds(0, P)]
  len_v[pl.ds(P, P)] = jnp.zeros((P,), jnp.int32)

  def lane0(j):
    return len_v[pl.ds(j, LANES)][0]

  begin = lax.fori_loop(0, p, lambda j, s: s + lane0(j), 0)
  length = lane0(p)

  mis = lax.rem(begin, 8)
  aligned = pl.multiple_of(begin - mis, 8)
  n = mis + length
  h = WIN // 2

  d0 = pltpu.make_async_copy(
      batch_hbm.at[SC_B, pl.ds(aligned, h), pl.ds(hoff, HALF)],
      buf.at[pl.ds(0, h)], sem0)
  d1 = pltpu.make_async_copy(
      batch_hbm.at[SC_B, pl.ds(aligned + h, h), pl.ds(hoff, HALF)],
      buf.at[pl.ds(h, h)], sem1)
  d0.start()

  @pl.when(n > h)
  def _():
    d1.start()

  zeros = tuple(jnp.zeros((LANES,), jnp.float32) for _ in range(NHALF))

  def body(r, accs):
    return tuple(accs[i] + buf[r, pl.ds(i * LANES, LANES)]
                 for i in range(NHALF))

  def accum_range(lo, hi, accs):
    cnt = jnp.maximum(hi - lo, 0)
    pairs = cnt // 2

    def body2(k, accs):
      r = lo + 2 * k
      return tuple(
          (accs[i] + buf[r, pl.ds(i * LANES, LANES)])
          + buf[r + 1, pl.ds(i * LANES, LANES)]
          for i in range(NHALF))

    accs = lax.fori_loop(0, pairs, body2, accs)
    return lax.fori_loop(lo + 2 * pairs, hi, body, accs)

  d0.wait()
  accs = accum_range(mis, jnp.minimum(n, h), zeros)

  @pl.when(n > h)
  def _():
    d1.wait()
  accs = accum_range(h, jnp.maximum(n, h), accs)
  denom = jnp.maximum(length, 1).astype(jnp.float32)
  for i in range(NHALF):
    outbuf[0, pl.ds(i * LANES, LANES)] = accs[i]

  def fix_chunk(i, _):
    v = outbuf[0, pl.ds(i * LANES, LANES)] / denom
    v = jnp.where(v == 0.0, jnp.full((LANES,), -1.0, jnp.float32), v)
    outbuf[0, pl.ds(i * LANES, LANES)] = v
    return 0
  lax.fori_loop(0, NHALF, fix_chunk, 0)
  pltpu.sync_copy(outbuf, out_sh.at[pl.ds(p_local, 1), pl.ds(hoff, HALF)])

  plsc.subcore_barrier()

  @pl.when(s_ax == 0)
  def _():
    pltpu.sync_copy(out_sh,
                    out_hbm.at[0, pl.ds(pl.multiple_of(c_ax * 8, 8), 8), :])


def _tc_body(len_ref, batch_ref, out_ref):
  pos = jax.lax.broadcasted_iota(jnp.int32, (1, S), 1)
  rows = []
  denoms = []
  cum = jnp.int32(0)
  one = jnp.ones((1, 1), jnp.float32)
  for p in range(P):
    ln = len_ref[0, 0, p]
    begin = cum
    cum = cum + ln
    rows.append(((pos >= begin) & (pos < cum)).astype(jnp.float32))
    denoms.append(one * jnp.maximum(ln, 1).astype(jnp.float32))
  mask = jnp.concatenate(rows, axis=0)
  denom = jnp.concatenate(denoms, axis=0)
  maskb = mask.astype(jnp.bfloat16)
  bf = batch_ref[...]
  bhi = bf.astype(jnp.bfloat16)
  blo = (bf - bhi.astype(jnp.float32)).astype(jnp.bfloat16)
  dn = (((1,), (0,)), ((), ()))
  acc = (jax.lax.dot_general(maskb, bhi, dimension_numbers=dn,
                             preferred_element_type=jnp.float32)
         + jax.lax.dot_general(maskb, blo, dimension_numbers=dn,
                               preferred_element_type=jnp.float32))
  res = acc / denom
  out_ref[...] = jnp.where(res == 0.0, -1.0, res)


@jax.jit
def kernel(batch, patch_lengths):
  lengths = patch_lengths
  if lengths.dtype != jnp.int32:
    lengths = lengths.astype(jnp.int32)

  mesh = plsc.VectorSubcoreMesh(core_axis_name="c", subcore_axis_name="s")
  sc_run = pl.kernel(
      _sc_body,
      out_type=jax.ShapeDtypeStruct((1, P, D), jnp.float32),
      mesh=mesh,
      scratch_types=[
          pltpu.VMEM((B, P), jnp.int32),
          pltpu.VMEM((2 * P,), jnp.int32),
          pltpu.VMEM((WIN, HALF), jnp.float32),
          pltpu.VMEM((1, HALF), jnp.float32),
          pltpu.VMEM_SHARED((8, D), jnp.float32),
          pltpu.SemaphoreType.DMA,
          pltpu.SemaphoreType.DMA,
      ],
  )
  sc_out = sc_run(batch, lengths)

  tc_fn = pl.pallas_call(
      lambda len_ref, batch_ref, out_ref: _tc_body(
          len_ref, batch_ref.at[0], out_ref.at[0]),
      grid=(B - 1,),
      in_specs=[
          pl.BlockSpec((1, 1, P), lambda b: (b, 0, 0),
                       memory_space=pltpu.SMEM),
          pl.BlockSpec((1, S, D), lambda b: (b, 0, 0)),
      ],
      out_specs=pl.BlockSpec((1, P, D), lambda b: (b, 0, 0)),
      out_shape=jax.ShapeDtypeStruct((B - 1, P, D), jnp.float32),
  )
  tc_out = tc_fn(lengths.reshape(B, 1, P), batch)

  return jnp.concatenate([tc_out, sc_out], axis=0)
